# baseline (speedup 1.0000x reference)
import functools

import jax
import jax.numpy as jnp
from jax import lax
from jax.experimental import pallas as pl
from jax.experimental.pallas import tpu as pltpu
from jax.experimental.pallas import tpu_sc as plsc

D_EMB = 1024
D_MODEL = 2048
N_TOK = 4 * 4096

NC = 2
NS = 16
NW = NC * NS
TW = N_TOK // NW
C = 16
NCHUNK = TW // C


def _proj_body(emb_ref, w_ref, b_ref, out_ref):
    out_ref[...] = (
        jnp.dot(emb_ref[...], w_ref[...], preferred_element_type=jnp.float32)
        + b_ref[...][None, :]
    )


def _compute_proj(emb_table, W, b):
    return pl.pallas_call(
        _proj_body,
        out_shape=jax.ShapeDtypeStruct((2, D_MODEL), jnp.float32),
    )(emb_table, W, b)


@functools.partial(
    pl.kernel,
    out_type=jax.ShapeDtypeStruct((N_TOK, D_MODEL), jnp.float32),
    mesh=plsc.VectorSubcoreMesh(core_axis_name="c", subcore_axis_name="s"),
    scratch_types=[
        pltpu.VMEM((TW,), jnp.int32),
        pltpu.VMEM((C, D_MODEL), jnp.float32),
        pltpu.SemaphoreType.DMA,
    ],
)
def _sc_gather(x_hbm, proj_hbm, out_hbm, idx_v, rows_v, sem):
    wid = lax.axis_index("s") * NC + lax.axis_index("c")
    base = wid * TW
    pltpu.sync_copy(x_hbm.at[pl.ds(base, TW)], idx_v)

    def body(ci, carry):
        cbase = ci * C
        pltpu.async_copy(
            proj_hbm.at[idx_v.at[pl.ds(cbase, C)]], rows_v, sem
        ).wait()
        pltpu.sync_copy(rows_v, out_hbm.at[pl.ds(base + cbase, C)])
        return carry

    lax.fori_loop(0, NCHUNK, body, 0)


def kernel(x, emb_table, W, b):
    proj = _compute_proj(emb_table, W, b)
    xf = x.reshape(-1).astype(jnp.int32)
    out = _sc_gather(xf, proj)
    return out.reshape(x.shape[0], x.shape[1], D_MODEL)


# speedup vs baseline: 1.0037x; 1.0037x over previous
import functools

import jax
import jax.numpy as jnp
from jax import lax
from jax.experimental import pallas as pl
from jax.experimental.pallas import tpu as pltpu
from jax.experimental.pallas import tpu_sc as plsc

D_EMB = 1024
D_MODEL = 2048
N_TOK = 4 * 4096

NC = 2
NS = 16
NW = NC * NS
TW = N_TOK // NW
C = 16
NCHUNK = TW // C
NBUF = 2
NG = NCHUNK // NBUF


def _proj_body(emb_ref, w_ref, b_ref, out_ref):
    out_ref[...] = (
        jnp.dot(emb_ref[...], w_ref[...], preferred_element_type=jnp.float32)
        + b_ref[...][None, :]
    )


def _compute_proj(emb_table, W, b):
    return pl.pallas_call(
        _proj_body,
        out_shape=jax.ShapeDtypeStruct((2, D_MODEL), jnp.float32),
    )(emb_table, W, b)


@functools.partial(
    pl.kernel,
    out_type=jax.ShapeDtypeStruct((N_TOK, D_MODEL), jnp.float32),
    mesh=plsc.VectorSubcoreMesh(core_axis_name="c", subcore_axis_name="s"),
    scratch_types=[
        pltpu.VMEM((TW,), jnp.int32),
        pltpu.VMEM((C, D_MODEL), jnp.float32),
        pltpu.VMEM((C, D_MODEL), jnp.float32),
        pltpu.SemaphoreType.DMA,
        pltpu.SemaphoreType.DMA,
        pltpu.SemaphoreType.DMA,
        pltpu.SemaphoreType.DMA,
    ],
)
def _sc_gather(x_hbm, proj_hbm, out_hbm, idx_v,
               b0, b1, gs0, gs1, ws0, ws1):
    cid = lax.axis_index("c")
    sid = lax.axis_index("s")
    wid = sid * NC + cid
    base = wid * TW
    pltpu.sync_copy(x_hbm.at[pl.ds(base, TW)], idx_v)

    bufs = (b0, b1)
    gsems = (gs0, gs1)
    wsems = (ws0, ws1)

    def start_gather(ci, p):
        pltpu.async_copy(
            proj_hbm.at[idx_v.at[pl.ds(ci * C, C)]], bufs[p], gsems[p]
        )

    def wait_gather(ci, p):
        pltpu.make_async_copy(
            proj_hbm.at[idx_v.at[pl.ds(ci * C, C)]], bufs[p], gsems[p]
        ).wait()

    def start_write(ci, p):
        pltpu.async_copy(bufs[p], out_hbm.at[pl.ds(base + ci * C, C)], wsems[p])

    def wait_write(ci, p):
        pltpu.make_async_copy(
            bufs[p], out_hbm.at[pl.ds(base + ci * C, C)], wsems[p]
        ).wait()

    start_gather(0, 0)

    def body(g, carry):
        ci0 = 2 * g
        wait_gather(ci0, 0)
        start_write(ci0, 0)

        @pl.when(g > 0)
        def _():
            wait_write(ci0 - 1, 1)

        start_gather(ci0 + 1, 1)

        wait_gather(ci0 + 1, 1)
        start_write(ci0 + 1, 1)
        wait_write(ci0, 0)

        @pl.when(g < NG - 1)
        def _():
            start_gather(ci0 + 2, 0)

        return carry

    lax.fori_loop(0, NG, body, 0)
    wait_write(NCHUNK - 1, 1)


def kernel(x, emb_table, W, b):
    proj = _compute_proj(emb_table, W, b)
    xf = x.reshape(-1).astype(jnp.int32)
    out = _sc_gather(xf, proj)
    return out.reshape(x.shape[0], x.shape[1], D_MODEL)
